# baseline (speedup 1.0000x reference)
import functools

import jax
import jax.numpy as jnp
from jax import lax
from jax.experimental import pallas as pl
from jax.experimental.pallas import tpu as pltpu
from jax.experimental.pallas import tpu_sc as plsc

NC = 2
NS = 16
L = 16
NW = NC * NS
B = 16384
D = 32
BPW = B // NW


def _sc_diff(user_emb, item_emb, user_ids, pos_ids, neg_ids):
    mesh = plsc.VectorSubcoreMesh(core_axis_name="c", subcore_axis_name="s")

    @functools.partial(
        pl.kernel,
        mesh=mesh,
        compiler_params=pltpu.CompilerParams(
            needs_layout_passes=False, use_tc_tiling_on_sc=False
        ),
        out_type=jax.ShapeDtypeStruct((B,), jnp.float32),
        scratch_types=[
            pltpu.VMEM((BPW,), jnp.int32),
            pltpu.VMEM((BPW,), jnp.int32),
            pltpu.VMEM((BPW,), jnp.int32),
            pltpu.VMEM((BPW, D), jnp.float32),
            pltpu.VMEM((BPW, D), jnp.float32),
            pltpu.VMEM((BPW, D), jnp.float32),
            pltpu.VMEM((BPW * L,), jnp.float32),
            pltpu.VMEM((BPW,), jnp.float32),
            pltpu.SemaphoreType.DMA,
        ],
    )
    def k(uemb, iemb, uids, pids, nids, out, iu, ip, inn, ur, pr, nr, a16, dv, sem):
        wid = lax.axis_index("s") * NC + lax.axis_index("c")
        base = wid * BPW
        pltpu.sync_copy(uids.at[pl.ds(base, BPW)], iu)
        pltpu.sync_copy(pids.at[pl.ds(base, BPW)], ip)
        pltpu.sync_copy(nids.at[pl.ds(base, BPW)], inn)
        cu = pltpu.async_copy(uemb.at[iu], ur, sem)
        cp = pltpu.async_copy(iemb.at[ip], pr, sem)
        cn = pltpu.async_copy(iemb.at[inn], nr, sem)
        cu.wait()
        cp.wait()
        cn.wait()

        def body_a(r, carry):
            d0 = pr[r, 0:16] - nr[r, 0:16]
            d1 = pr[r, 16:32] - nr[r, 16:32]
            a16[pl.ds(r * L, L)] = ur[r, 0:16] * d0 + ur[r, 16:32] * d1
            return carry

        lax.fori_loop(0, BPW, body_a, 0, unroll=4)

        iota = lax.iota(jnp.int32, L)

        def body_b(t, carry):
            flat = (t * L + iota) * L
            s = plsc.load_gather(a16, [flat])
            for l in range(1, L):
                s = s + plsc.load_gather(a16, [flat + l])
            dv[pl.ds(t * L, L)] = s
            return carry

        lax.fori_loop(0, BPW // L, body_b, 0)
        pltpu.sync_copy(dv, out.at[pl.ds(base, BPW)])

    return k(user_emb, item_emb, user_ids, pos_ids, neg_ids)


def _tc_loss_kernel(x_ref, o_ref):
    o_ref[0, 0] = -jnp.sum(jax.nn.log_sigmoid(x_ref[:, :]))


def _tc_loss(diff):
    x = diff.reshape(B // 128, 128)
    res = pl.pallas_call(
        _tc_loss_kernel,
        out_shape=jax.ShapeDtypeStruct((1, 1), jnp.float32),
        out_specs=pl.BlockSpec(memory_space=pltpu.SMEM),
    )(x)
    return res[0, 0]


def kernel(user_emb, item_emb, user_ids, pos_ids, neg_ids):
    diff = _sc_diff(user_emb, item_emb, user_ids, pos_ids, neg_ids)
    return _tc_loss(diff)
